# Initial kernel scaffold; baseline (speedup 1.0000x reference)
import jax
import jax.numpy as jnp
from jax.experimental import pallas as pl


def kernel(u, i, edge_index, user_table, entity_table, W1, b1, W2, b2):
    raise NotImplementedError("write your pallas kernel here")


# speedup vs baseline: 11.9506x; 11.9506x over previous
import functools

import jax
import jax.numpy as jnp
from jax import lax
from jax.experimental import pallas as pl
from jax.experimental.pallas import tpu as pltpu
from jax.experimental.pallas import tpu_sc as plsc

NN = 10000
NPAD = 10240
D = 128
E = 320000
BATCH = 4096
NC, NS = 2, 16
NW = NC * NS
CH = 128
CPT = 79
EPAD = NW * CPT * CH
RPT = NPAD // NS

_mesh = plsc.VectorSubcoreMesh(core_axis_name="c", subcore_axis_name="s")


def _zero_rows(ref, nrows, ncols):
    z16 = jnp.zeros((16,), jnp.float32)

    def body(j, _):
        for k in range(ncols // 16):
            ref[j, pl.ds(k * 16, 16)] = z16
        return 0

    lax.fori_loop(0, nrows, body, 0)


@functools.partial(
    pl.kernel,
    out_type=[
        jax.ShapeDtypeStruct((NC, NPAD, 16), jnp.float32),
        jax.ShapeDtypeStruct((BATCH, D), jnp.float32),
    ],
    mesh=_mesh,
    scratch_types=[
        pltpu.VMEM((CPT, CH), jnp.int32),
        pltpu.VMEM((CH, 16), jnp.float32),
        pltpu.VMEM((CH,), jnp.int32),
        pltpu.VMEM((CH, D), jnp.float32),
        pltpu.VMEM_SHARED((NPAD, 16), jnp.float32),
        pltpu.SemaphoreType.DMA,
    ],
)
def _deg_users_kernel(dstp, u_idx, user_table, deg_out, users_out,
                      idxd, ones, idxu, urows, dacc, sem):
    c = lax.axis_index("c")
    s = lax.axis_index("s")
    wid = c * NS + s

    pltpu.sync_copy(dstp.at[wid], idxd)

    _zero_rows(ones, CH, 16)
    for t in range(RPT // CH):
        pltpu.sync_copy(ones, dacc.at[pl.ds(s * RPT + t * CH, CH)])

    pltpu.sync_copy(u_idx.at[wid], idxu)
    pltpu.async_copy(user_table.at[idxu], urows, sem).wait()
    pltpu.sync_copy(urows, users_out.at[pl.ds(wid * CH, CH)])

    e0 = jnp.where(lax.iota(jnp.int32, 16) == 0, 1.0, 0.0)

    def fill(j, _):
        ones[j] = e0
        return 0

    lax.fori_loop(0, CH, fill, 0)
    plsc.subcore_barrier()

    def hist(j, _):
        pltpu.sync_copy(ones, dacc.at[idxd.at[j]], add=True)
        return 0

    lax.fori_loop(0, CPT, hist, 0)
    plsc.subcore_barrier()

    pltpu.sync_copy(dacc.at[pl.ds(s * RPT, RPT)],
                    deg_out.at[c, pl.ds(s * RPT, RPT)])


@functools.partial(
    pl.kernel,
    out_type=jax.ShapeDtypeStruct((NC, NPAD, D), jnp.float32),
    mesh=_mesh,
    scratch_types=[
        pltpu.VMEM((CPT, CH), jnp.int32),
        pltpu.VMEM((CPT, CH), jnp.int32),
        pltpu.VMEM((CH, D), jnp.float32),
        pltpu.VMEM_SHARED((NPAD, D), jnp.float32),
        pltpu.SemaphoreType.DMA,
    ],
)
def _edge_scatter_kernel(srcp, dstp, z, out, idxs, idxd, rows, acc, sem):
    c = lax.axis_index("c")
    s = lax.axis_index("s")
    wid = c * NS + s

    pltpu.sync_copy(srcp.at[wid], idxs)
    pltpu.sync_copy(dstp.at[wid], idxd)

    _zero_rows(rows, CH, D)
    for t in range(RPT // CH):
        pltpu.sync_copy(rows, acc.at[pl.ds(s * RPT + t * CH, CH)])
    plsc.subcore_barrier()

    def body(j, _):
        pltpu.async_copy(z.at[idxs.at[j]], rows, sem).wait()
        pltpu.sync_copy(rows, acc.at[idxd.at[j]], add=True)
        return 0

    lax.fori_loop(0, CPT, body, 0)
    plsc.subcore_barrier()

    pltpu.sync_copy(acc.at[pl.ds(s * RPT, RPT)],
                    out.at[c, pl.ds(s * RPT, RPT)])


@functools.partial(
    pl.kernel,
    out_type=jax.ShapeDtypeStruct((BATCH, D), jnp.float32),
    mesh=_mesh,
    scratch_types=[
        pltpu.VMEM((CH,), jnp.int32),
        pltpu.VMEM((CH, D), jnp.float32),
        pltpu.SemaphoreType.DMA,
    ],
)
def _gather_kernel(idx_hbm, table, out, idxv, rows, sem):
    wid = lax.axis_index("c") * NS + lax.axis_index("s")
    pltpu.sync_copy(idx_hbm.at[wid], idxv)
    pltpu.async_copy(table.at[idxv], rows, sem).wait()
    pltpu.sync_copy(rows, out.at[pl.ds(wid * CH, CH)])


def _dis(deg_ref):
    d = deg_ref[0, :, 0:1] + deg_ref[1, :, 0:1] + 1.0
    return lax.rsqrt(d)


def _renorm_block(x):
    n = jnp.sqrt(jnp.sum(x * x, axis=1, keepdims=True))
    return x * jnp.where(n > 1.0, 1.0 / (n + 1e-7), 1.0)


def _mm(a, b):
    return lax.dot_general(a, b, (((1,), (0,)), ((), ())),
                           preferred_element_type=jnp.float32,
                           precision=lax.Precision.HIGHEST)


_BR = 2048
_GRID = NPAD // _BR


def _k1_body(ent_ref, deg_ref, w_ref, out_ref):
    x = _renorm_block(ent_ref[...])
    out_ref[...] = _mm(x, w_ref[...]) * _dis(deg_ref)


def _k3_body(z_ref, s_ref, deg_ref, b_ref, w_ref, out_ref):
    dis = _dis(deg_ref)
    h = (z_ref[...] + s_ref[0] + s_ref[1]) * dis + b_ref[...]
    out_ref[...] = _mm(jnp.maximum(h, 0.0), w_ref[...]) * dis


def _k5_body(z_ref, s_ref, deg_ref, b_ref, out_ref):
    out_ref[...] = (z_ref[...] + s_ref[0] + s_ref[1]) * _dis(deg_ref) + b_ref[...]


def _k7_body(u_ref, it_ref, out_ref):
    us = _renorm_block(u_ref[...])
    uv = jnp.sum(us * it_ref[...], axis=1, keepdims=True)
    out_ref[...] = jax.nn.sigmoid(uv)


def _row_spec(r3=False):
    if r3:
        return pl.BlockSpec((NC, _BR, D), lambda r: (0, r, 0))
    return pl.BlockSpec((_BR, D), lambda r: (r, 0))


_DEG_SPEC = pl.BlockSpec((NC, _BR, 16), lambda r: (0, r, 0))
_W_SPEC = pl.BlockSpec((D, D), lambda r: (0, 0))
_B_SPEC = pl.BlockSpec((1, D), lambda r: (0, 0))


def kernel(u, i, edge_index, user_table, entity_table, W1, b1, W2, b2):
    src = edge_index[0].astype(jnp.int32)
    dst = edge_index[1].astype(jnp.int32)
    pad = jnp.full((EPAD - E,), NN, jnp.int32)
    srcp = jnp.concatenate([src, pad]).reshape(NW, CPT, CH)
    dstp = jnp.concatenate([dst, pad]).reshape(NW, CPT, CH)
    ent = jnp.concatenate(
        [entity_table, jnp.zeros((NPAD - NN, D), jnp.float32)], axis=0)
    u2 = u.astype(jnp.int32).reshape(NW, CH)
    i2 = i.astype(jnp.int32).reshape(NW, CH)
    b1r = b1.reshape(1, D)
    b2r = b2.reshape(1, D)

    deg2, users_raw = _deg_users_kernel(dstp, u2, user_table)

    z1 = pl.pallas_call(
        _k1_body, grid=(_GRID,),
        in_specs=[_row_spec(), _DEG_SPEC, _W_SPEC],
        out_specs=_row_spec(),
        out_shape=jax.ShapeDtypeStruct((NPAD, D), jnp.float32),
    )(ent, deg2, W1)

    s1 = _edge_scatter_kernel(srcp, dstp, z1)

    z2 = pl.pallas_call(
        _k3_body, grid=(_GRID,),
        in_specs=[_row_spec(), _row_spec(True), _DEG_SPEC, _B_SPEC, _W_SPEC],
        out_specs=_row_spec(),
        out_shape=jax.ShapeDtypeStruct((NPAD, D), jnp.float32),
    )(z1, s1, deg2, b1r, W2)

    s2 = _edge_scatter_kernel(srcp, dstp, z2)

    x2 = pl.pallas_call(
        _k5_body, grid=(_GRID,),
        in_specs=[_row_spec(), _row_spec(True), _DEG_SPEC, _B_SPEC],
        out_specs=_row_spec(),
        out_shape=jax.ShapeDtypeStruct((NPAD, D), jnp.float32),
    )(z2, s2, deg2, b2r)

    items = _gather_kernel(i2, x2)

    uv = pl.pallas_call(
        _k7_body, grid=(2,),
        in_specs=[pl.BlockSpec((BATCH // 2, D), lambda r: (r, 0)),
                  pl.BlockSpec((BATCH // 2, D), lambda r: (r, 0))],
        out_specs=pl.BlockSpec((BATCH // 2, 1), lambda r: (r, 0)),
        out_shape=jax.ShapeDtypeStruct((BATCH, 1), jnp.float32),
    )(users_raw, items)

    return uv.reshape(BATCH)
